# baseline (speedup 1.0000x reference)
import jax
import jax.numpy as jnp
from jax.experimental import pallas as pl
from jax.experimental.pallas import tpu as pltpu

_SIZE = 65536
_SHIFT = 8192
_ROWS = 64
_LANE = 128
_OUTV = _ROWS * _SIZE // _LANE
_RPB = _SIZE // _LANE
_ZPB = (_SIZE - _SHIFT) // _LANE
_XPB = _SHIFT // _LANE


def _body(x_ref, o_ref):
    o_ref[: _ZPB] = jnp.zeros((_ZPB, _LANE), jnp.float32)
    o_ref[_ZPB :] = x_ref[...]


def kernel(x):
    xv = x.reshape(-1, _LANE)
    out = pl.pallas_call(
        _body,
        grid=(_ROWS,),
        in_specs=[pl.BlockSpec((_XPB, _LANE), lambda i: (i, 0))],
        out_specs=pl.BlockSpec((_RPB, _LANE), lambda i: (i, 0)),
        out_shape=jax.ShapeDtypeStruct((_OUTV, _LANE), jnp.float32),
        compiler_params=pltpu.CompilerParams(
            dimension_semantics=("arbitrary",),
        ),
    )(xv)
    return out.reshape(x.shape[:-1] + (_SIZE,))


# speedup vs baseline: 1.7896x; 1.7896x over previous
import jax
import jax.numpy as jnp
from jax.experimental import pallas as pl
from jax.experimental.pallas import tpu as pltpu

_SIZE = 65536
_SHIFT = 8192
_ZLEN = _SIZE - _SHIFT
_ROWS = 64


def _body(x_hbm, o_hbm, zbuf, xbuf, zsem, isem, osem):
    icp = pltpu.make_async_copy(x_hbm, xbuf, isem)
    icp.start()
    zbuf[...] = jnp.zeros_like(zbuf)
    zcps = [
        pltpu.make_async_copy(
            zbuf, o_hbm.at[pl.ds(r * _SIZE, _ZLEN)], zsem)
        for r in range(_ROWS)
    ]
    for c in zcps:
        c.start()
    icp.wait()
    wcps = [
        pltpu.make_async_copy(
            xbuf.at[pl.ds(r * _SHIFT, _SHIFT)],
            o_hbm.at[pl.ds(r * _SIZE + _ZLEN, _SHIFT)], osem)
        for r in range(_ROWS)
    ]
    for c in wcps:
        c.start()
    for c in zcps:
        c.wait()
    for c in wcps:
        c.wait()


def kernel(x):
    xf = x.reshape(_ROWS * _SHIFT)
    out = pl.pallas_call(
        _body,
        in_specs=[pl.BlockSpec(memory_space=pl.ANY)],
        out_specs=pl.BlockSpec(memory_space=pl.ANY),
        out_shape=jax.ShapeDtypeStruct((_ROWS * _SIZE,), jnp.float32),
        scratch_shapes=[
            pltpu.VMEM((_ZLEN,), jnp.float32),
            pltpu.VMEM((_ROWS * _SHIFT,), jnp.float32),
            pltpu.SemaphoreType.DMA,
            pltpu.SemaphoreType.DMA,
            pltpu.SemaphoreType.DMA,
        ],
    )(xf)
    return out.reshape(x.shape[:-1] + (_SIZE,))
